# baseline (speedup 1.0000x reference)
import functools

import jax
import jax.numpy as jnp
from jax import lax
from jax.experimental import pallas as pl
from jax.experimental.pallas import tpu as pltpu
from jax.experimental.pallas import tpu_sc as plsc

N = 50000
E = 800000
H = 64
NC = 2
NS = 16
HALF = N // NC
STRIPE = 1568
SPR = NS * STRIPE
TRASH = HALF
EB = 80
EPT = E // NS
SB = 2000
SUB = SB // EB
NCH = EPT // SB
RING = 2

_MESH = plsc.VectorSubcoreMesh(core_axis_name="c", subcore_axis_name="s")


@functools.partial(
    pl.kernel,
    out_type=jax.ShapeDtypeStruct((NC, SPR, H), jnp.float32),
    mesh=_MESH,
    scratch_types=[
        pltpu.VMEM((2, SB), jnp.int32),
        pltpu.VMEM((2, SB), jnp.int32),
        pltpu.VMEM((SUB, EB), jnp.int32),
        pltpu.VMEM((SUB, EB), jnp.int32),
        pltpu.VMEM((RING, EB, H), jnp.float32),
        pltpu.VMEM_SHARED((SPR, H), jnp.float32),
        pltpu.SemaphoreType.DMA,
        pltpu.SemaphoreType.DMA,
        pltpu.SemaphoreType.DMA,
    ],
    compiler_params=pltpu.CompilerParams(use_tc_tiling_on_sc=False),
)
def _sc_scatter(g_hbm, src_hbm, dst_hbm, zeros_hbm, out_hbm,
                sidx, didx, lbuf, msrc, rows, acc, gsem, ssem, isem):
    c = lax.axis_index("c")
    s = lax.axis_index("s")
    base = c * HALF

    def stage(ci):
        e0 = pl.multiple_of(s * EPT + ci * SB, 8)
        b = lax.rem(ci, 2)
        pltpu.async_copy(src_hbm.at[pl.ds(e0, SB)], sidx.at[b], isem)
        pltpu.async_copy(dst_hbm.at[pl.ds(e0, SB)], didx.at[b], isem)

    def wait_stage():
        pltpu.make_async_copy(src_hbm.at[pl.ds(0, SB)], sidx.at[0],
                              isem).wait()
        pltpu.make_async_copy(dst_hbm.at[pl.ds(0, SB)], didx.at[0],
                              isem).wait()

    def wait_gather():
        pltpu.make_async_copy(g_hbm.at[msrc.at[0]], rows.at[0], gsem).wait()

    def start_scatter(j):
        pltpu.async_copy(rows.at[lax.rem(j, RING)], acc.at[lbuf.at[j]],
                         ssem, add=True)

    def wait_scatter():
        pltpu.make_async_copy(rows.at[0], acc.at[lbuf.at[0]], ssem).wait()

    stage(0)
    pltpu.sync_copy(zeros_hbm, acc.at[pl.ds(s * STRIPE, STRIPE)])
    plsc.subcore_barrier()

    def chunk(ci, carry):
        b = lax.rem(ci, 2)
        wait_stage()

        @pl.when(ci + 1 < NCH)
        def _():
            stage(ci + 1)

        def start_gather(j):
            pltpu.async_copy(g_hbm.at[msrc.at[j]],
                             rows.at[lax.rem(j, RING)], gsem)

        def compute_loc(j):
            for kk in range(EB // 16):
                d = didx[b, pl.ds(j * EB + kk * 16, 16)]
                l = d - base
                ok = (l >= 0) & (l < HALF)
                lbuf[j, pl.ds(kk * 16, 16)] = jnp.where(ok, l, TRASH)
                sv = sidx[b, pl.ds(j * EB + kk * 16, 16)]
                msrc[j, pl.ds(kk * 16, 16)] = jnp.where(ok, sv, 0)

        compute_loc(0)
        start_gather(0)

        def inner(j, carry):
            @pl.when(j >= RING - 1)
            def _():
                wait_scatter()
            start_gather(j + 1)
            compute_loc(j + 1)
            wait_gather()
            start_scatter(j)
            return carry

        lax.fori_loop(0, SUB - 1, inner, 0)
        wait_gather()
        start_scatter(SUB - 1)
        for _ in range(RING):
            wait_scatter()
        return carry

    lax.fori_loop(0, NCH, chunk, 0)
    plsc.subcore_barrier()
    pltpu.sync_copy(acc.at[pl.ds(s * STRIPE, STRIPE)],
                    out_hbm.at[c].at[pl.ds(s * STRIPE, STRIPE)])


@functools.partial(
    pl.kernel,
    out_type=jax.ShapeDtypeStruct((NC, SPR, 16), jnp.float32),
    mesh=_MESH,
    scratch_types=[
        pltpu.VMEM((2, SB), jnp.int32),
        pltpu.VMEM((SUB, EB), jnp.int32),
        pltpu.VMEM((EB, 16), jnp.float32),
        pltpu.VMEM_SHARED((SPR, 16), jnp.float32),
        pltpu.SemaphoreType.DMA,
        pltpu.SemaphoreType.DMA,
    ],
    compiler_params=pltpu.CompilerParams(use_tc_tiling_on_sc=False),
)
def _sc_degree(dst_hbm, zeros_hbm, ones_hbm, out_hbm, didx, lbuf, ones_v, acc,
               ssem, isem):
    c = lax.axis_index("c")
    s = lax.axis_index("s")
    base = c * HALF

    def stage(ci):
        e0 = pl.multiple_of(s * EPT + ci * SB, 8)
        pltpu.async_copy(dst_hbm.at[pl.ds(e0, SB)], didx.at[lax.rem(ci, 2)],
                         isem)

    def wait_stage():
        pltpu.make_async_copy(dst_hbm.at[pl.ds(0, SB)], didx.at[0],
                              isem).wait()

    stage(0)
    pltpu.sync_copy(zeros_hbm, acc.at[pl.ds(s * STRIPE, STRIPE)])
    pltpu.sync_copy(ones_hbm, ones_v)
    plsc.subcore_barrier()

    def chunk(ci, carry):
        b = lax.rem(ci, 2)
        wait_stage()

        @pl.when(ci + 1 < NCH)
        def _():
            stage(ci + 1)

        def fire(j, carry):
            for kk in range(EB // 16):
                d = didx[b, pl.ds(j * EB + kk * 16, 16)]
                l = d - base
                ok = (l >= 0) & (l < HALF)
                lbuf[j, pl.ds(kk * 16, 16)] = jnp.where(ok, l, TRASH)
            pltpu.async_copy(ones_v, acc.at[lbuf.at[j]], ssem, add=True)
            return carry

        lax.fori_loop(0, SUB, fire, 0)

        def drain(j, carry):
            pltpu.make_async_copy(ones_v, acc.at[lbuf.at[0]], ssem).wait()
            return carry

        lax.fori_loop(0, SUB, drain, 0)
        return carry

    lax.fori_loop(0, NCH, chunk, 0)
    plsc.subcore_barrier()
    pltpu.sync_copy(acc.at[pl.ds(s * STRIPE, STRIPE)],
                    out_hbm.at[c].at[pl.ds(s * STRIPE, STRIPE)])


RB = 1000
NRB = N // RB
HB = HALF // RB


def _ln(x, gamma, beta):
    m = jnp.mean(x, axis=-1, keepdims=True)
    v = jnp.mean((x - m) ** 2, axis=-1, keepdims=True)
    return (x - m) * lax.rsqrt(v + 1e-5) * gamma + beta


def _dinv_of(dg):
    deg = dg[0][:, :1] + 1.0
    return lax.rsqrt(jnp.maximum(deg, 1e-12))


def _rep(shape):
    return pl.BlockSpec(shape, lambda j: tuple(0 for _ in shape))


def _rows(f):
    return pl.BlockSpec((RB, f), lambda j: (j, 0))


def _sc_rows(f):
    return pl.BlockSpec((1, RB, f), lambda j: (j // HB, j % HB, 0))


def _encoder(x, W_enc, b_enc, gamma, beta, deg_sc, Wc0):
    def body(x_ref, We, be, ga, bb, dg, W0, h_ref, g_ref):
        h = jnp.dot(x_ref[...], We[...], preferred_element_type=jnp.float32)
        h = jnp.maximum(h + be[...], 0.0)
        h = _ln(h, ga[...], bb[...])
        h_ref[...] = h
        g_ref[...] = _dinv_of(dg) * jnp.dot(
            h, W0[...], preferred_element_type=jnp.float32)

    return pl.pallas_call(
        body,
        grid=(NRB,),
        in_specs=[_rows(2), _rep((2, H)), _rep((1, H)), _rep((1, H)),
                  _rep((1, H)), _sc_rows(16), _rep((H, H))],
        out_specs=[_rows(H), _rows(H)],
        out_shape=[jax.ShapeDtypeStruct((N, H), jnp.float32),
                   jax.ShapeDtypeStruct((N, H), jnp.float32)],
    )(x, W_enc, b_enc, gamma, beta, deg_sc, Wc0)


def _layer(s_sc, gprev, h, deg_sc, bci, gamma, beta, Wnext):
    def body(s_ref, gp, h_ref, dg, bc, ga, bb, Wn, ho, go):
        dinv = _dinv_of(dg)
        t = jnp.maximum(dinv * (s_ref[0] + gp[...]) + bc[...], 0.0)
        hn = _ln(t + h_ref[...], ga[...], bb[...])
        ho[...] = hn
        go[...] = dinv * jnp.dot(hn, Wn[...], preferred_element_type=jnp.float32)

    return pl.pallas_call(
        body,
        grid=(NRB,),
        in_specs=[_sc_rows(H), _rows(H), _rows(H), _sc_rows(16),
                  _rep((1, H)), _rep((1, H)), _rep((1, H)), _rep((H, H))],
        out_specs=[_rows(H), _rows(H)],
        out_shape=[jax.ShapeDtypeStruct((N, H), jnp.float32),
                   jax.ShapeDtypeStruct((N, H), jnp.float32)],
    )(s_sc, gprev, h, deg_sc, bci, gamma, beta, Wnext)


def _final(s_sc, gprev, h, deg_sc, bci, gamma, beta, Wf1, bf1, Wf2, bf2):
    def body(s_ref, gp, h_ref, dg, bc, ga, bb, W1, b1, W2, b2, y_ref):
        dinv = _dinv_of(dg)
        t = jnp.maximum(dinv * (s_ref[0] + gp[...]) + bc[...], 0.0)
        hn = _ln(t + h_ref[...], ga[...], bb[...])
        f = jnp.maximum(
            jnp.dot(hn, W1[...], preferred_element_type=jnp.float32) + b1[...],
            0.0)
        y_ref[...] = jnp.tanh(
            jnp.dot(f, W2[...], preferred_element_type=jnp.float32) + b2[...])

    return pl.pallas_call(
        body,
        grid=(NRB,),
        in_specs=[_sc_rows(H), _rows(H), _rows(H), _sc_rows(16),
                  _rep((1, H)), _rep((1, H)), _rep((1, H)),
                  _rep((H, 32)), _rep((1, 32)), _rep((32, 2)), _rep((1, 2))],
        out_specs=_rows(2),
        out_shape=jax.ShapeDtypeStruct((N, 2), jnp.float32),
    )(s_sc, gprev, h, deg_sc, bci, gamma, beta, Wf1, bf1, Wf2, bf2)


def kernel(x, edge_index, W_enc, b_enc, gamma, beta, Wc, bc, Wf1, bf1, Wf2, bf2):
    src = edge_index[0]
    dst = edge_index[1]
    zeros64 = jnp.zeros((STRIPE, H), jnp.float32)
    zeros16 = jnp.zeros((STRIPE, 16), jnp.float32)
    ones16 = jnp.ones((EB, 16), jnp.float32)
    gamma2 = gamma.reshape(1, H)
    beta2 = beta.reshape(1, H)

    deg_sc = _sc_degree(dst, zeros16, ones16)
    h, g = _encoder(x, W_enc, b_enc.reshape(1, H), gamma2, beta2,
                    deg_sc, Wc[0])
    for i in range(4):
        s_sc = _sc_scatter(g, src, dst, zeros64)
        if i < 3:
            h, g = _layer(s_sc, g, h, deg_sc, bc[i].reshape(1, H),
                          gamma2, beta2, Wc[i + 1])
        else:
            y = _final(s_sc, g, h, deg_sc, bc[3].reshape(1, H), gamma2, beta2,
                       Wf1, bf1.reshape(1, 32), Wf2, bf2.reshape(1, 2))
    return y


# speedup vs baseline: 28.4865x; 28.4865x over previous
import functools

import jax
import jax.numpy as jnp
from jax import lax
from jax.experimental import pallas as pl
from jax.experimental.pallas import tpu as pltpu
from jax.experimental.pallas import tpu_sc as plsc

N = 50000
E = 800000
H = 64
NC = 2
NS = 16
HALF = N // NC
STRIPE = 1568
SPR = NS * STRIPE
TRASH = HALF
EB = 80
EPT = E // NS
SB = 2000
SUB = SB // EB
NCH = EPT // SB
RING = 2

_MESH = plsc.VectorSubcoreMesh(core_axis_name="c", subcore_axis_name="s")


@functools.partial(
    pl.kernel,
    out_type=jax.ShapeDtypeStruct((NC, SPR, H), jnp.float32),
    mesh=_MESH,
    scratch_types=[
        pltpu.VMEM((2, SB), jnp.int32),
        pltpu.VMEM((2, SB), jnp.int32),
        pltpu.VMEM((SUB, EB), jnp.int32),
        pltpu.VMEM((RING, EB, H), jnp.float32),
        pltpu.VMEM_SHARED((SPR, H), jnp.float32),
        pltpu.SemaphoreType.DMA,
        pltpu.SemaphoreType.DMA,
        pltpu.SemaphoreType.DMA,
    ],
    compiler_params=pltpu.CompilerParams(use_tc_tiling_on_sc=False),
)
def _sc_scatter(g_hbm, src_hbm, dst_hbm, zeros_hbm, out_hbm,
                sidx, didx, lbuf, rows, acc, gsem, ssem, isem):
    c = lax.axis_index("c")
    s = lax.axis_index("s")
    base = c * HALF

    def stage(ci):
        e0 = pl.multiple_of(s * EPT + ci * SB, 8)
        b = lax.rem(ci, 2)
        pltpu.async_copy(src_hbm.at[pl.ds(e0, SB)], sidx.at[b], isem)
        pltpu.async_copy(dst_hbm.at[pl.ds(e0, SB)], didx.at[b], isem)

    def wait_stage():
        pltpu.make_async_copy(src_hbm.at[pl.ds(0, SB)], sidx.at[0],
                              isem).wait()
        pltpu.make_async_copy(dst_hbm.at[pl.ds(0, SB)], didx.at[0],
                              isem).wait()

    def wait_gather():
        pltpu.make_async_copy(g_hbm.at[sidx.at[0].at[pl.ds(0, EB)]],
                              rows.at[0], gsem).wait()

    def start_scatter(j):
        pltpu.async_copy(rows.at[lax.rem(j, RING)], acc.at[lbuf.at[j]],
                         ssem, add=True)

    def wait_scatter():
        pltpu.make_async_copy(rows.at[0], acc.at[lbuf.at[0]], ssem).wait()

    stage(0)
    pltpu.sync_copy(zeros_hbm, acc.at[pl.ds(s * STRIPE, STRIPE)])
    plsc.subcore_barrier()

    def chunk(ci, carry):
        b = lax.rem(ci, 2)
        wait_stage()

        @pl.when(ci + 1 < NCH)
        def _():
            stage(ci + 1)

        def start_gather(j):
            pltpu.async_copy(
                g_hbm.at[sidx.at[b].at[pl.ds(pl.multiple_of(j * EB, 8), EB)]],
                rows.at[lax.rem(j, RING)], gsem)

        def compute_loc(j):
            ii = lax.iota(jnp.int32, 16)
            for kk in range(EB // 16):
                d = didx[b, pl.ds(j * EB + kk * 16, 16)]
                l = d - base
                ok = (l >= 0) & (l < HALF)
                lbuf[j, pl.ds(kk * 16, 16)] = jnp.where(ok, l, TRASH + ii)

        compute_loc(0)
        start_gather(0)

        def inner(j, carry):
            @pl.when(j >= RING - 1)
            def _():
                wait_scatter()
            start_gather(j + 1)
            compute_loc(j + 1)
            wait_gather()
            start_scatter(j)
            return carry

        lax.fori_loop(0, SUB - 1, inner, 0)
        wait_gather()
        start_scatter(SUB - 1)
        for _ in range(RING):
            wait_scatter()
        return carry

    lax.fori_loop(0, NCH, chunk, 0)
    plsc.subcore_barrier()
    pltpu.sync_copy(acc.at[pl.ds(s * STRIPE, STRIPE)],
                    out_hbm.at[c].at[pl.ds(s * STRIPE, STRIPE)])


@functools.partial(
    pl.kernel,
    out_type=jax.ShapeDtypeStruct((NC, SPR, 16), jnp.float32),
    mesh=_MESH,
    scratch_types=[
        pltpu.VMEM((2, SB), jnp.int32),
        pltpu.VMEM((SUB, EB), jnp.int32),
        pltpu.VMEM((EB, 16), jnp.float32),
        pltpu.VMEM_SHARED((SPR, 16), jnp.float32),
        pltpu.SemaphoreType.DMA,
        pltpu.SemaphoreType.DMA,
    ],
    compiler_params=pltpu.CompilerParams(use_tc_tiling_on_sc=False),
)
def _sc_degree(dst_hbm, zeros_hbm, ones_hbm, out_hbm, didx, lbuf, ones_v, acc,
               ssem, isem):
    c = lax.axis_index("c")
    s = lax.axis_index("s")
    base = c * HALF

    def stage(ci):
        e0 = pl.multiple_of(s * EPT + ci * SB, 8)
        pltpu.async_copy(dst_hbm.at[pl.ds(e0, SB)], didx.at[lax.rem(ci, 2)],
                         isem)

    def wait_stage():
        pltpu.make_async_copy(dst_hbm.at[pl.ds(0, SB)], didx.at[0],
                              isem).wait()

    stage(0)
    pltpu.sync_copy(zeros_hbm, acc.at[pl.ds(s * STRIPE, STRIPE)])
    pltpu.sync_copy(ones_hbm, ones_v)
    plsc.subcore_barrier()

    def chunk(ci, carry):
        b = lax.rem(ci, 2)
        wait_stage()

        @pl.when(ci + 1 < NCH)
        def _():
            stage(ci + 1)

        def fire(j, carry):
            ii = lax.iota(jnp.int32, 16)
            for kk in range(EB // 16):
                d = didx[b, pl.ds(j * EB + kk * 16, 16)]
                l = d - base
                ok = (l >= 0) & (l < HALF)
                lbuf[j, pl.ds(kk * 16, 16)] = jnp.where(ok, l, TRASH + ii)
            pltpu.async_copy(ones_v, acc.at[lbuf.at[j]], ssem, add=True)
            return carry

        lax.fori_loop(0, SUB, fire, 0)

        def drain(j, carry):
            pltpu.make_async_copy(ones_v, acc.at[lbuf.at[0]], ssem).wait()
            return carry

        lax.fori_loop(0, SUB, drain, 0)
        return carry

    lax.fori_loop(0, NCH, chunk, 0)
    plsc.subcore_barrier()
    pltpu.sync_copy(acc.at[pl.ds(s * STRIPE, STRIPE)],
                    out_hbm.at[c].at[pl.ds(s * STRIPE, STRIPE)])


RB = 1000
NRB = N // RB
HB = HALF // RB


def _ln(x, gamma, beta):
    m = jnp.mean(x, axis=-1, keepdims=True)
    v = jnp.mean((x - m) ** 2, axis=-1, keepdims=True)
    return (x - m) * lax.rsqrt(v + 1e-5) * gamma + beta


def _dinv_of(dg):
    deg = dg[0][:, :1] + 1.0
    return lax.rsqrt(jnp.maximum(deg, 1e-12))


def _rep(shape):
    return pl.BlockSpec(shape, lambda j: tuple(0 for _ in shape))


def _rows(f):
    return pl.BlockSpec((RB, f), lambda j: (j, 0))


def _sc_rows(f):
    return pl.BlockSpec((1, RB, f), lambda j: (j // HB, j % HB, 0))


def _encoder(x, W_enc, b_enc, gamma, beta, deg_sc, Wc0):
    def body(x_ref, We, be, ga, bb, dg, W0, h_ref, g_ref):
        h = jnp.dot(x_ref[...], We[...], preferred_element_type=jnp.float32)
        h = jnp.maximum(h + be[...], 0.0)
        h = _ln(h, ga[...], bb[...])
        h_ref[...] = h
        g_ref[...] = _dinv_of(dg) * jnp.dot(
            h, W0[...], preferred_element_type=jnp.float32)

    return pl.pallas_call(
        body,
        grid=(NRB,),
        in_specs=[_rows(2), _rep((2, H)), _rep((1, H)), _rep((1, H)),
                  _rep((1, H)), _sc_rows(16), _rep((H, H))],
        out_specs=[_rows(H), _rows(H)],
        out_shape=[jax.ShapeDtypeStruct((N, H), jnp.float32),
                   jax.ShapeDtypeStruct((N, H), jnp.float32)],
    )(x, W_enc, b_enc, gamma, beta, deg_sc, Wc0)


def _layer(s_sc, gprev, h, deg_sc, bci, gamma, beta, Wnext):
    def body(s_ref, gp, h_ref, dg, bc, ga, bb, Wn, ho, go):
        dinv = _dinv_of(dg)
        t = jnp.maximum(dinv * (s_ref[0] + gp[...]) + bc[...], 0.0)
        hn = _ln(t + h_ref[...], ga[...], bb[...])
        ho[...] = hn
        go[...] = dinv * jnp.dot(hn, Wn[...], preferred_element_type=jnp.float32)

    return pl.pallas_call(
        body,
        grid=(NRB,),
        in_specs=[_sc_rows(H), _rows(H), _rows(H), _sc_rows(16),
                  _rep((1, H)), _rep((1, H)), _rep((1, H)), _rep((H, H))],
        out_specs=[_rows(H), _rows(H)],
        out_shape=[jax.ShapeDtypeStruct((N, H), jnp.float32),
                   jax.ShapeDtypeStruct((N, H), jnp.float32)],
    )(s_sc, gprev, h, deg_sc, bci, gamma, beta, Wnext)


def _final(s_sc, gprev, h, deg_sc, bci, gamma, beta, Wf1, bf1, Wf2, bf2):
    def body(s_ref, gp, h_ref, dg, bc, ga, bb, W1, b1, W2, b2, y_ref):
        dinv = _dinv_of(dg)
        t = jnp.maximum(dinv * (s_ref[0] + gp[...]) + bc[...], 0.0)
        hn = _ln(t + h_ref[...], ga[...], bb[...])
        f = jnp.maximum(
            jnp.dot(hn, W1[...], preferred_element_type=jnp.float32) + b1[...],
            0.0)
        y_ref[...] = jnp.tanh(
            jnp.dot(f, W2[...], preferred_element_type=jnp.float32) + b2[...])

    return pl.pallas_call(
        body,
        grid=(NRB,),
        in_specs=[_sc_rows(H), _rows(H), _rows(H), _sc_rows(16),
                  _rep((1, H)), _rep((1, H)), _rep((1, H)),
                  _rep((H, 32)), _rep((1, 32)), _rep((32, 2)), _rep((1, 2))],
        out_specs=_rows(2),
        out_shape=jax.ShapeDtypeStruct((N, 2), jnp.float32),
    )(s_sc, gprev, h, deg_sc, bci, gamma, beta, Wf1, bf1, Wf2, bf2)


def kernel(x, edge_index, W_enc, b_enc, gamma, beta, Wc, bc, Wf1, bf1, Wf2, bf2):
    src = edge_index[0]
    dst = edge_index[1]
    zeros64 = jnp.zeros((STRIPE, H), jnp.float32)
    zeros16 = jnp.zeros((STRIPE, 16), jnp.float32)
    ones16 = jnp.ones((EB, 16), jnp.float32)
    gamma2 = gamma.reshape(1, H)
    beta2 = beta.reshape(1, H)

    deg_sc = _sc_degree(dst, zeros16, ones16)
    h, g = _encoder(x, W_enc, b_enc.reshape(1, H), gamma2, beta2,
                    deg_sc, Wc[0])
    for i in range(4):
        s_sc = _sc_scatter(g, src, dst, zeros64)
        if i < 3:
            h, g = _layer(s_sc, g, h, deg_sc, bc[i].reshape(1, H),
                          gamma2, beta2, Wc[i + 1])
        else:
            y = _final(s_sc, g, h, deg_sc, bc[3].reshape(1, H), gamma2, beta2,
                       Wf1, bf1.reshape(1, 32), Wf2, bf2.reshape(1, 2))
    return y


# speedup vs baseline: 28.5262x; 1.0014x over previous
import functools

import jax
import jax.numpy as jnp
from jax import lax
from jax.experimental import pallas as pl
from jax.experimental.pallas import tpu as pltpu
from jax.experimental.pallas import tpu_sc as plsc

N = 50000
E = 800000
H = 64
NC = 2
NS = 16
HALF = N // NC
STRIPE = 1584
SPR = NS * STRIPE
TRASH = HALF
EB = 80
EPT = E // NS
SB = 2000
SUB = SB // EB
NCH = EPT // SB
RING = 2

_MESH = plsc.VectorSubcoreMesh(core_axis_name="c", subcore_axis_name="s")


@functools.partial(
    pl.kernel,
    out_type=jax.ShapeDtypeStruct((NC, SPR, H), jnp.float32),
    mesh=_MESH,
    scratch_types=[
        pltpu.VMEM((2, SB), jnp.int32),
        pltpu.VMEM((2, SB), jnp.int32),
        pltpu.VMEM((SUB, EB), jnp.int32),
        pltpu.VMEM((RING, EB, H), jnp.float32),
        pltpu.VMEM_SHARED((SPR, H), jnp.float32),
        pltpu.SemaphoreType.DMA,
        pltpu.SemaphoreType.DMA,
        pltpu.SemaphoreType.DMA,
    ],
    compiler_params=pltpu.CompilerParams(use_tc_tiling_on_sc=False),
)
def _sc_scatter(g_hbm, src_hbm, dst_hbm, zeros_hbm, out_hbm,
                sidx, didx, lbuf, rows, acc, gsem, ssem, isem):
    c = lax.axis_index("c")
    s = lax.axis_index("s")
    base = c * HALF

    def stage(ci):
        e0 = pl.multiple_of(s * EPT + ci * SB, 8)
        b = lax.rem(ci, 2)
        pltpu.async_copy(src_hbm.at[pl.ds(e0, SB)], sidx.at[b], isem)
        pltpu.async_copy(dst_hbm.at[pl.ds(e0, SB)], didx.at[b], isem)

    def wait_stage():
        pltpu.make_async_copy(src_hbm.at[pl.ds(0, SB)], sidx.at[0],
                              isem).wait()
        pltpu.make_async_copy(dst_hbm.at[pl.ds(0, SB)], didx.at[0],
                              isem).wait()

    def wait_gather():
        pltpu.make_async_copy(g_hbm.at[sidx.at[0].at[pl.ds(0, EB)]],
                              rows.at[0], gsem).wait()

    def start_scatter(j):
        pltpu.async_copy(rows.at[lax.rem(j, RING)], acc.at[lbuf.at[j]],
                         ssem, add=True)

    def wait_scatter():
        pltpu.make_async_copy(rows.at[0], acc.at[lbuf.at[0]], ssem).wait()

    stage(0)
    pltpu.sync_copy(zeros_hbm, acc.at[pl.ds(s * STRIPE, STRIPE)])
    plsc.subcore_barrier()

    def chunk(ci, carry):
        b = lax.rem(ci, 2)
        wait_stage()

        @pl.when(ci + 1 < NCH)
        def _():
            stage(ci + 1)

        def start_gather(j):
            pltpu.async_copy(
                g_hbm.at[sidx.at[b].at[pl.ds(pl.multiple_of(j * EB, 8), EB)]],
                rows.at[lax.rem(j, RING)], gsem)

        trash = TRASH + s * 16

        def compute_loc(j):
            ii = lax.iota(jnp.int32, 16)
            for kk in range(EB // 16):
                d = didx[b, pl.ds(j * EB + kk * 16, 16)]
                l = d - base
                ok = (l >= 0) & (l < HALF)
                lbuf[j, pl.ds(kk * 16, 16)] = jnp.where(ok, l, trash + ii)

        compute_loc(0)
        start_gather(0)

        def inner(j, carry):
            @pl.when(j >= RING - 1)
            def _():
                wait_scatter()
            start_gather(j + 1)
            compute_loc(j + 1)
            wait_gather()
            start_scatter(j)
            return carry

        lax.fori_loop(0, SUB - 1, inner, 0)
        wait_gather()
        start_scatter(SUB - 1)
        for _ in range(RING):
            wait_scatter()
        return carry

    lax.fori_loop(0, NCH, chunk, 0)
    plsc.subcore_barrier()
    pltpu.sync_copy(acc.at[pl.ds(s * STRIPE, STRIPE)],
                    out_hbm.at[c].at[pl.ds(s * STRIPE, STRIPE)])


@functools.partial(
    pl.kernel,
    out_type=jax.ShapeDtypeStruct((NC, SPR, 16), jnp.float32),
    mesh=_MESH,
    scratch_types=[
        pltpu.VMEM((2, SB), jnp.int32),
        pltpu.VMEM((SUB, EB), jnp.int32),
        pltpu.VMEM((EB, 16), jnp.float32),
        pltpu.VMEM_SHARED((SPR, 16), jnp.float32),
        pltpu.SemaphoreType.DMA,
        pltpu.SemaphoreType.DMA,
    ],
    compiler_params=pltpu.CompilerParams(use_tc_tiling_on_sc=False),
)
def _sc_degree(dst_hbm, zeros_hbm, ones_hbm, out_hbm, didx, lbuf, ones_v, acc,
               ssem, isem):
    c = lax.axis_index("c")
    s = lax.axis_index("s")
    base = c * HALF

    def stage(ci):
        e0 = pl.multiple_of(s * EPT + ci * SB, 8)
        pltpu.async_copy(dst_hbm.at[pl.ds(e0, SB)], didx.at[lax.rem(ci, 2)],
                         isem)

    def wait_stage():
        pltpu.make_async_copy(dst_hbm.at[pl.ds(0, SB)], didx.at[0],
                              isem).wait()

    stage(0)
    pltpu.sync_copy(zeros_hbm, acc.at[pl.ds(s * STRIPE, STRIPE)])
    pltpu.sync_copy(ones_hbm, ones_v)
    plsc.subcore_barrier()

    def chunk(ci, carry):
        b = lax.rem(ci, 2)
        wait_stage()

        @pl.when(ci + 1 < NCH)
        def _():
            stage(ci + 1)

        trash = TRASH + s * 16

        def fire(j, carry):
            ii = lax.iota(jnp.int32, 16)
            for kk in range(EB // 16):
                d = didx[b, pl.ds(j * EB + kk * 16, 16)]
                l = d - base
                ok = (l >= 0) & (l < HALF)
                lbuf[j, pl.ds(kk * 16, 16)] = jnp.where(ok, l, trash + ii)
            pltpu.async_copy(ones_v, acc.at[lbuf.at[j]], ssem, add=True)
            return carry

        lax.fori_loop(0, SUB, fire, 0)

        def drain(j, carry):
            pltpu.make_async_copy(ones_v, acc.at[lbuf.at[0]], ssem).wait()
            return carry

        lax.fori_loop(0, SUB, drain, 0)
        return carry

    lax.fori_loop(0, NCH, chunk, 0)
    plsc.subcore_barrier()
    pltpu.sync_copy(acc.at[pl.ds(s * STRIPE, STRIPE)],
                    out_hbm.at[c].at[pl.ds(s * STRIPE, STRIPE)])


RB = 1000
NRB = N // RB
HB = HALF // RB


def _ln(x, gamma, beta):
    m = jnp.mean(x, axis=-1, keepdims=True)
    v = jnp.mean((x - m) ** 2, axis=-1, keepdims=True)
    return (x - m) * lax.rsqrt(v + 1e-5) * gamma + beta


def _dinv_of(dg):
    deg = dg[0][:, :1] + 1.0
    return lax.rsqrt(jnp.maximum(deg, 1e-12))


def _rep(shape):
    return pl.BlockSpec(shape, lambda j: tuple(0 for _ in shape))


def _rows(f):
    return pl.BlockSpec((RB, f), lambda j: (j, 0))


def _sc_rows(f):
    return pl.BlockSpec((1, RB, f), lambda j: (j // HB, j % HB, 0))


def _encoder(x, W_enc, b_enc, gamma, beta, deg_sc, Wc0):
    def body(x_ref, We, be, ga, bb, dg, W0, h_ref, g_ref):
        h = jnp.dot(x_ref[...], We[...], preferred_element_type=jnp.float32)
        h = jnp.maximum(h + be[...], 0.0)
        h = _ln(h, ga[...], bb[...])
        h_ref[...] = h
        g_ref[...] = _dinv_of(dg) * jnp.dot(
            h, W0[...], preferred_element_type=jnp.float32)

    return pl.pallas_call(
        body,
        grid=(NRB,),
        in_specs=[_rows(2), _rep((2, H)), _rep((1, H)), _rep((1, H)),
                  _rep((1, H)), _sc_rows(16), _rep((H, H))],
        out_specs=[_rows(H), _rows(H)],
        out_shape=[jax.ShapeDtypeStruct((N, H), jnp.float32),
                   jax.ShapeDtypeStruct((N, H), jnp.float32)],
    )(x, W_enc, b_enc, gamma, beta, deg_sc, Wc0)


def _layer(s_sc, gprev, h, deg_sc, bci, gamma, beta, Wnext):
    def body(s_ref, gp, h_ref, dg, bc, ga, bb, Wn, ho, go):
        dinv = _dinv_of(dg)
        t = jnp.maximum(dinv * (s_ref[0] + gp[...]) + bc[...], 0.0)
        hn = _ln(t + h_ref[...], ga[...], bb[...])
        ho[...] = hn
        go[...] = dinv * jnp.dot(hn, Wn[...], preferred_element_type=jnp.float32)

    return pl.pallas_call(
        body,
        grid=(NRB,),
        in_specs=[_sc_rows(H), _rows(H), _rows(H), _sc_rows(16),
                  _rep((1, H)), _rep((1, H)), _rep((1, H)), _rep((H, H))],
        out_specs=[_rows(H), _rows(H)],
        out_shape=[jax.ShapeDtypeStruct((N, H), jnp.float32),
                   jax.ShapeDtypeStruct((N, H), jnp.float32)],
    )(s_sc, gprev, h, deg_sc, bci, gamma, beta, Wnext)


def _final(s_sc, gprev, h, deg_sc, bci, gamma, beta, Wf1, bf1, Wf2, bf2):
    def body(s_ref, gp, h_ref, dg, bc, ga, bb, W1, b1, W2, b2, y_ref):
        dinv = _dinv_of(dg)
        t = jnp.maximum(dinv * (s_ref[0] + gp[...]) + bc[...], 0.0)
        hn = _ln(t + h_ref[...], ga[...], bb[...])
        f = jnp.maximum(
            jnp.dot(hn, W1[...], preferred_element_type=jnp.float32) + b1[...],
            0.0)
        y_ref[...] = jnp.tanh(
            jnp.dot(f, W2[...], preferred_element_type=jnp.float32) + b2[...])

    return pl.pallas_call(
        body,
        grid=(NRB,),
        in_specs=[_sc_rows(H), _rows(H), _rows(H), _sc_rows(16),
                  _rep((1, H)), _rep((1, H)), _rep((1, H)),
                  _rep((H, 32)), _rep((1, 32)), _rep((32, 2)), _rep((1, 2))],
        out_specs=_rows(2),
        out_shape=jax.ShapeDtypeStruct((N, 2), jnp.float32),
    )(s_sc, gprev, h, deg_sc, bci, gamma, beta, Wf1, bf1, Wf2, bf2)


def kernel(x, edge_index, W_enc, b_enc, gamma, beta, Wc, bc, Wf1, bf1, Wf2, bf2):
    src = edge_index[0]
    dst = edge_index[1]
    zeros64 = jnp.zeros((STRIPE, H), jnp.float32)
    zeros16 = jnp.zeros((STRIPE, 16), jnp.float32)
    ones16 = jnp.ones((EB, 16), jnp.float32)
    gamma2 = gamma.reshape(1, H)
    beta2 = beta.reshape(1, H)

    deg_sc = _sc_degree(dst, zeros16, ones16)
    h, g = _encoder(x, W_enc, b_enc.reshape(1, H), gamma2, beta2,
                    deg_sc, Wc[0])
    for i in range(4):
        s_sc = _sc_scatter(g, src, dst, zeros64)
        if i < 3:
            h, g = _layer(s_sc, g, h, deg_sc, bc[i].reshape(1, H),
                          gamma2, beta2, Wc[i + 1])
        else:
            y = _final(s_sc, g, h, deg_sc, bc[3].reshape(1, H), gamma2, beta2,
                       Wf1, bf1.reshape(1, 32), Wf2, bf2.reshape(1, 2))
    return y
